# baseline (speedup 1.0000x reference)
import functools

import jax
import jax.numpy as jnp
from jax import lax
from jax.experimental import pallas as pl
from jax.experimental.pallas import tpu as pltpu
from jax.experimental.pallas import tpu_sc as plsc

N_NODES = 100000
D_FEAT = 128
K_RPR = 16
NODEVEC = 200
BATCH = 4096
NEG = 256
TOTAL = 2 * BATCH + NEG

_NC = 2
_NS = 16
_NW = _NC * _NS
_NA = BATCH // _NW
_NB = NEG // _NW
_CH = 8
_DEPTH = 4


def _agg_body(ti_hbm, tl_hbm, ns_hbm, feat_hbm, rprm_hbm, rpra_hbm,
              weighted_out, ids_v, arg_v, wts_v, feat_v, wrow_v, rsem, fsem):
  wid = lax.axis_index("s") * _NC + lax.axis_index("c")

  def do_pass(ids_hbm, n, out_base):
    base = out_base + wid * n
    pltpu.sync_copy(ids_hbm.at[pl.ds(wid * n, n)], ids_v.at[pl.ds(0, n)])
    nchunks = n // _CH

    def fire(c):
      idv = ids_v[pl.ds(c * _CH, 16)]
      for i in range(_CH):
        tid = idv[i]
        pltpu.async_copy(rpra_hbm.at[tid], arg_v.at[c * _CH + i], rsem)
        pltpu.async_copy(rprm_hbm.at[tid], wts_v.at[c * _CH + i], rsem)

    def drain_rpr(c):
      for i in range(_CH):
        pltpu.make_async_copy(rpra_hbm.at[0], arg_v.at[c * _CH + i],
                              rsem).wait()
        pltpu.make_async_copy(rprm_hbm.at[0], wts_v.at[c * _CH + i],
                              rsem).wait()

    fire(0)

    def p1_chunk(c, carry):
      @pl.when(c + 1 < nchunks)
      def _():
        fire(c + 1)
      drain_rpr(c)
      return carry

    lax.fori_loop(0, nchunks, p1_chunk, 0)

    def fire_feat(i):
      pltpu.async_copy(feat_hbm.at[arg_v[i, :]],
                       feat_v.at[lax.rem(i, _DEPTH)], fsem)

    for d in range(_DEPTH - 1):
      fire_feat(d)

    def body(i, carry):
      slot = lax.rem(i, _DEPTH)

      @pl.when(i + _DEPTH - 1 < n)
      def _():
        fire_feat(i + _DEPTH - 1)

      pltpu.make_async_copy(feat_hbm.at[pl.ds(0, K_RPR)], feat_v.at[slot],
                            fsem).wait()
      accs = [jnp.zeros((16,), jnp.float32) for _ in range(D_FEAT // 16)]
      wrow = wts_v[i, :]
      for k in range(K_RPR):
        wk = wrow[k]
        for j in range(D_FEAT // 16):
          accs[j] = accs[j] + wk * feat_v[slot, k, pl.ds(16 * j, 16)]
      for j in range(D_FEAT // 16):
        wrow_v[i, pl.ds(16 * j, 16)] = accs[j]
      return carry

    lax.fori_loop(0, n, body, 0)
    pltpu.sync_copy(wrow_v.at[pl.ds(0, n)], weighted_out.at[pl.ds(base, n)])

  do_pass(ti_hbm, _NA, 0)
  do_pass(tl_hbm, _NA, BATCH)
  do_pass(ns_hbm, _NB, 2 * BATCH)


@functools.cache
def _make_agg():
  return pl.kernel(
      _agg_body,
      out_type=jax.ShapeDtypeStruct((TOTAL, D_FEAT), jnp.float32),
      mesh=plsc.VectorSubcoreMesh(core_axis_name="c", subcore_axis_name="s"),
      scratch_types=[
          pltpu.VMEM((_NA + 16,), jnp.int32),
          pltpu.VMEM((_NA, K_RPR), jnp.int32),
          pltpu.VMEM((_NA, K_RPR), jnp.float32),
          pltpu.VMEM((_DEPTH, K_RPR, D_FEAT), jnp.float32),
          pltpu.VMEM((_NA, D_FEAT), jnp.float32),
          pltpu.SemaphoreType.DMA,
          pltpu.SemaphoreType.DMA,
      ],
  )


def _row_pass(wid, ids_hbm, table_hbm, gathered_out, ids_v, out_v, esem,
              n, out_base):
  base = out_base + wid * n
  pltpu.sync_copy(ids_hbm.at[pl.ds(wid * n, n)], ids_v.at[pl.ds(0, n)])
  nchunks = n // _CH

  def fire(c):
    idv = ids_v[pl.ds(c * _CH, 16)]
    for i in range(_CH):
      pltpu.async_copy(table_hbm.at[idv[i]], out_v.at[c * _CH + i], esem)

  fire(0)

  def chunk(c, carry):
    @pl.when(c + 1 < nchunks)
    def _():
      fire(c + 1)
    for i in range(_CH):
      pltpu.make_async_copy(table_hbm.at[0], out_v.at[c * _CH + i],
                            esem).wait()
    return carry

  lax.fori_loop(0, nchunks, chunk, 0)
  pltpu.sync_copy(out_v.at[pl.ds(0, n)], gathered_out.at[pl.ds(base, n)])


def _emb1_body(w_hbm, ti_hbm, emb_hbm, g1_out, ids_v, out_v, esem):
  del w_hbm
  wid = lax.axis_index("s") * _NC + lax.axis_index("c")
  _row_pass(wid, ti_hbm, emb_hbm, g1_out, ids_v, out_v, esem, _NA, 0)


def _emb2_body(w_hbm, tl_hbm, ns_hbm, nce_hbm, g2_out, ids_v, out_v, esem):
  del w_hbm
  wid = lax.axis_index("s") * _NC + lax.axis_index("c")
  _row_pass(wid, tl_hbm, nce_hbm, g2_out, ids_v, out_v, esem, _NA, 0)
  _row_pass(wid, ns_hbm, nce_hbm, g2_out, ids_v, out_v, esem, _NB, BATCH)


def _emb_scratch():
  return [
      pltpu.VMEM((_NA + 16,), jnp.int32),
      pltpu.VMEM((_NA, NODEVEC), jnp.float32),
      pltpu.SemaphoreType.DMA,
  ]


@functools.cache
def _make_emb1():
  return pl.kernel(
      _emb1_body,
      out_type=jax.ShapeDtypeStruct((BATCH, NODEVEC), jnp.float32),
      mesh=plsc.VectorSubcoreMesh(core_axis_name="c", subcore_axis_name="s"),
      scratch_types=_emb_scratch(),
  )


@functools.cache
def _make_emb2():
  return pl.kernel(
      _emb2_body,
      out_type=jax.ShapeDtypeStruct((BATCH + NEG, NODEVEC), jnp.float32),
      mesh=plsc.VectorSubcoreMesh(core_axis_name="c", subcore_axis_name="s"),
      scratch_types=_emb_scratch(),
  )


def _log_sig(x):
  return jnp.log(jax.nn.sigmoid(x) + 0.001)


def _tc_body(w_ref, g1_ref, g2_ref, wa_ref, tia_ref, tla_ref, nsa_ref,
             loss_ref):
  wagg = wa_ref[...]
  f32 = jnp.float32
  tif = jnp.dot(w_ref[0:BATCH, :], wagg, preferred_element_type=f32)
  tlf = jnp.dot(w_ref[BATCH:2 * BATCH, :], wagg, preferred_element_type=f32)
  nsf = jnp.dot(w_ref[2 * BATCH:TOTAL, :], wagg, preferred_element_type=f32)
  embed = g1_ref[...]
  truew = g2_ref[0:BATCH, :]
  falsew = g2_ref[BATCH:BATCH + NEG, :]
  tia_ref[...] = tif + embed
  tla_ref[...] = tlf + truew
  nsa_ref[...] = nsf + falsew
  s1 = jnp.sum(_log_sig(jnp.sum(tif * tlf, axis=1)))
  s3 = jnp.sum(_log_sig(jnp.sum(embed * truew, axis=1)))
  s5 = jnp.sum(_log_sig(jnp.sum(embed * tlf, axis=1)))
  s7 = jnp.sum(_log_sig(jnp.sum(truew * tif, axis=1)))
  sum_tif = jnp.sum(tif, axis=0)
  sum_embed = jnp.sum(embed, axis=0)
  sum_truew = jnp.sum(truew, axis=0)
  sum_nsf = jnp.sum(nsf, axis=0)
  sum_falsew = jnp.sum(falsew, axis=0)
  p2 = _log_sig(-jnp.sum(sum_tif * sum_nsf))
  p4 = _log_sig(-jnp.sum(sum_embed * sum_falsew))
  p6 = _log_sig(-jnp.sum(sum_embed * sum_nsf))
  p8 = _log_sig(-jnp.sum(sum_truew * sum_nsf))
  b = jnp.float32(BATCH)
  total = (1.5 * (s1 + b * p2) + 0.75 * (s3 + b * p4)
           + 1.5 * (s5 + b * p6) + 1.5 * (s7 + b * p8))
  loss_ref[0, 0] = -total / b


_tc_call = pl.pallas_call(
    _tc_body,
    out_shape=[
        jax.ShapeDtypeStruct((BATCH, NODEVEC), jnp.float32),
        jax.ShapeDtypeStruct((BATCH, NODEVEC), jnp.float32),
        jax.ShapeDtypeStruct((NEG, NODEVEC), jnp.float32),
        jax.ShapeDtypeStruct((1, 1), jnp.float32),
    ],
    out_specs=[
        pl.BlockSpec(memory_space=pltpu.VMEM),
        pl.BlockSpec(memory_space=pltpu.VMEM),
        pl.BlockSpec(memory_space=pltpu.VMEM),
        pl.BlockSpec(memory_space=pltpu.SMEM),
    ],
)


def kernel(train_inputs, train_labels, neg_samples, features, rpr_matrix,
           rpr_arg, embeddings, nce_weights, W_agg):
  weighted = _make_agg()(train_inputs, train_labels, neg_samples,
                         features, rpr_matrix, rpr_arg)
  g1 = _make_emb1()(weighted, train_inputs, embeddings)
  g2 = _make_emb2()(weighted, train_labels, neg_samples, nce_weights)
  tia, tla, nsa, loss = _tc_call(weighted, g1, g2, W_agg)
  return (loss[0, 0], tia, tla, nsa)


# speedup vs baseline: 1.0383x; 1.0383x over previous
import functools

import jax
import jax.numpy as jnp
from jax import lax
from jax.experimental import pallas as pl
from jax.experimental.pallas import tpu as pltpu
from jax.experimental.pallas import tpu_sc as plsc

N_NODES = 100000
D_FEAT = 128
K_RPR = 16
NODEVEC = 200
BATCH = 4096
NEG = 256
TOTAL = 2 * BATCH + NEG

_NC = 2
_NS = 16
_NW = _NC * _NS
_NA = BATCH // _NW
_NB = NEG // _NW
_CH = 8
_DEPTH = 4


def _agg_body(ti_hbm, tl_hbm, ns_hbm, feat_hbm, rprm_hbm, rpra_hbm,
              weighted_out, ids_v, arg_v, wts_v, feat_v, wrow_v, rsem, fsem):
  wid = lax.axis_index("s") * _NC + lax.axis_index("c")

  def do_pass(ids_hbm, n, out_base):
    base = out_base + wid * n
    pltpu.sync_copy(ids_hbm.at[pl.ds(wid * n, n)], ids_v.at[pl.ds(0, n)])
    nchunks = n // _CH

    def fire(c):
      idv = ids_v[pl.ds(c * _CH, 16)]
      for i in range(_CH):
        tid = idv[i]
        pltpu.async_copy(rpra_hbm.at[tid], arg_v.at[c * _CH + i], rsem)
        pltpu.async_copy(rprm_hbm.at[tid], wts_v.at[c * _CH + i], rsem)

    def drain_rpr(c):
      for i in range(_CH):
        pltpu.make_async_copy(rpra_hbm.at[0], arg_v.at[c * _CH + i],
                              rsem).wait()
        pltpu.make_async_copy(rprm_hbm.at[0], wts_v.at[c * _CH + i],
                              rsem).wait()

    fire(0)

    def p1_chunk(c, carry):
      @pl.when(c + 1 < nchunks)
      def _():
        fire(c + 1)
      drain_rpr(c)
      return carry

    lax.fori_loop(0, nchunks, p1_chunk, 0)

    def fire_feat(i):
      pltpu.async_copy(feat_hbm.at[arg_v[i, :]],
                       feat_v.at[lax.rem(i, _DEPTH)], fsem)

    for d in range(_DEPTH - 1):
      fire_feat(d)

    def body(i, carry):
      slot = lax.rem(i, _DEPTH)

      @pl.when(i + _DEPTH - 1 < n)
      def _():
        fire_feat(i + _DEPTH - 1)

      pltpu.make_async_copy(feat_hbm.at[pl.ds(0, K_RPR)], feat_v.at[slot],
                            fsem).wait()
      accs = [jnp.zeros((16,), jnp.float32) for _ in range(D_FEAT // 16)]
      wrow = wts_v[i, :]
      for k in range(K_RPR):
        wk = wrow[k]
        for j in range(D_FEAT // 16):
          accs[j] = accs[j] + wk * feat_v[slot, k, pl.ds(16 * j, 16)]
      for j in range(D_FEAT // 16):
        wrow_v[i, pl.ds(16 * j, 16)] = accs[j]
      return carry

    lax.fori_loop(0, n, body, 0)
    pltpu.sync_copy(wrow_v.at[pl.ds(0, n)], weighted_out.at[pl.ds(base, n)])

  do_pass(ti_hbm, _NA, 0)
  do_pass(tl_hbm, _NA, BATCH)
  do_pass(ns_hbm, _NB, 2 * BATCH)


@functools.cache
def _make_agg():
  return pl.kernel(
      _agg_body,
      out_type=jax.ShapeDtypeStruct((TOTAL, D_FEAT), jnp.float32),
      mesh=plsc.VectorSubcoreMesh(core_axis_name="c", subcore_axis_name="s"),
      scratch_types=[
          pltpu.VMEM((_NA + 16,), jnp.int32),
          pltpu.VMEM((_NA, K_RPR), jnp.int32),
          pltpu.VMEM((_NA, K_RPR), jnp.float32),
          pltpu.VMEM((_DEPTH, K_RPR, D_FEAT), jnp.float32),
          pltpu.VMEM((_NA, D_FEAT), jnp.float32),
          pltpu.SemaphoreType.DMA,
          pltpu.SemaphoreType.DMA,
      ],
  )


def _emb_body(w_hbm, ti_hbm, tl_hbm, ns_hbm, emb_hbm, nce_hbm, gathered_out,
              ids_v, out_v, esem):
  del w_hbm
  wid = lax.axis_index("s") * _NC + lax.axis_index("c")

  def do_pass(ids_hbm, table_hbm, n, out_base):
    base = out_base + wid * n
    pltpu.sync_copy(ids_hbm.at[pl.ds(wid * n, n)], ids_v.at[pl.ds(0, n)])
    nchunks = n // _CH

    def fire(c):
      idv = ids_v[pl.ds(c * _CH, 16)]
      for i in range(_CH):
        pltpu.async_copy(table_hbm.at[idv[i]], out_v.at[c * _CH + i], esem)

    fire(0)
    if nchunks > 1:
      fire(1)

    def chunk(c, carry):
      @pl.when(c + 2 < nchunks)
      def _():
        fire(c + 2)
      for i in range(_CH):
        pltpu.make_async_copy(table_hbm.at[0], out_v.at[c * _CH + i],
                              esem).wait()
      return carry

    lax.fori_loop(0, nchunks, chunk, 0)
    pltpu.sync_copy(out_v.at[pl.ds(0, n)], gathered_out.at[pl.ds(base, n)])

  do_pass(ti_hbm, emb_hbm, _NA, 0)
  do_pass(tl_hbm, nce_hbm, _NA, BATCH)
  do_pass(ns_hbm, nce_hbm, _NB, 2 * BATCH)


@functools.cache
def _make_emb():
  return pl.kernel(
      _emb_body,
      out_type=jax.ShapeDtypeStruct((TOTAL, NODEVEC), jnp.float32),
      mesh=plsc.VectorSubcoreMesh(core_axis_name="c", subcore_axis_name="s"),
      scratch_types=[
          pltpu.VMEM((_NA + 16,), jnp.int32),
          pltpu.VMEM((_NA, NODEVEC), jnp.float32),
          pltpu.SemaphoreType.DMA,
      ],
  )


def _log_sig(x):
  return jnp.log(jax.nn.sigmoid(x) + 0.001)


def _tc_body(w_ref, g_ref, wa_ref, tia_ref, tla_ref, nsa_ref, loss_ref):
  wagg = wa_ref[...]
  f32 = jnp.float32
  tif = jnp.dot(w_ref[0:BATCH, :], wagg, preferred_element_type=f32)
  tlf = jnp.dot(w_ref[BATCH:2 * BATCH, :], wagg, preferred_element_type=f32)
  nsf = jnp.dot(w_ref[2 * BATCH:TOTAL, :], wagg, preferred_element_type=f32)
  embed = g_ref[0:BATCH, :]
  truew = g_ref[BATCH:2 * BATCH, :]
  falsew = g_ref[2 * BATCH:TOTAL, :]
  tia_ref[...] = tif + embed
  tla_ref[...] = tlf + truew
  nsa_ref[...] = nsf + falsew
  s1 = jnp.sum(_log_sig(jnp.sum(tif * tlf, axis=1)))
  s3 = jnp.sum(_log_sig(jnp.sum(embed * truew, axis=1)))
  s5 = jnp.sum(_log_sig(jnp.sum(embed * tlf, axis=1)))
  s7 = jnp.sum(_log_sig(jnp.sum(truew * tif, axis=1)))
  sum_tif = jnp.sum(tif, axis=0)
  sum_embed = jnp.sum(embed, axis=0)
  sum_truew = jnp.sum(truew, axis=0)
  sum_nsf = jnp.sum(nsf, axis=0)
  sum_falsew = jnp.sum(falsew, axis=0)
  p2 = _log_sig(-jnp.sum(sum_tif * sum_nsf))
  p4 = _log_sig(-jnp.sum(sum_embed * sum_falsew))
  p6 = _log_sig(-jnp.sum(sum_embed * sum_nsf))
  p8 = _log_sig(-jnp.sum(sum_truew * sum_nsf))
  b = jnp.float32(BATCH)
  total = (1.5 * (s1 + b * p2) + 0.75 * (s3 + b * p4)
           + 1.5 * (s5 + b * p6) + 1.5 * (s7 + b * p8))
  loss_ref[0, 0] = -total / b


_tc_call = pl.pallas_call(
    _tc_body,
    out_shape=[
        jax.ShapeDtypeStruct((BATCH, NODEVEC), jnp.float32),
        jax.ShapeDtypeStruct((BATCH, NODEVEC), jnp.float32),
        jax.ShapeDtypeStruct((NEG, NODEVEC), jnp.float32),
        jax.ShapeDtypeStruct((1, 1), jnp.float32),
    ],
    out_specs=[
        pl.BlockSpec(memory_space=pltpu.VMEM),
        pl.BlockSpec(memory_space=pltpu.VMEM),
        pl.BlockSpec(memory_space=pltpu.VMEM),
        pl.BlockSpec(memory_space=pltpu.SMEM),
    ],
)


def kernel(train_inputs, train_labels, neg_samples, features, rpr_matrix,
           rpr_arg, embeddings, nce_weights, W_agg):
  weighted = _make_agg()(train_inputs, train_labels, neg_samples,
                         features, rpr_matrix, rpr_arg)
  gathered = _make_emb()(weighted, train_inputs, train_labels, neg_samples,
                         embeddings, nce_weights)
  tia, tla, nsa, loss = _tc_call(weighted, gathered, W_agg)
  return (loss[0, 0], tia, tla, nsa)
